# Initial kernel scaffold; baseline (speedup 1.0000x reference)
import jax
import jax.numpy as jnp
from jax.experimental import pallas as pl


def kernel(x, edge_index, edge_attr, W1m, b1m, W1e, b1e, W1s, b1s, W2m, b2m, W2e, b2e, W2s, b2s):
    raise NotImplementedError("write your pallas kernel here")


# speedup vs baseline: 4.1141x; 4.1141x over previous
import functools

import jax
import jax.numpy as jnp
from jax import lax
from jax.experimental import pallas as pl
from jax.experimental.pallas import tpu as pltpu
from jax.experimental.pallas import tpu_sc as plsc

NC = 2
NS = 16
CH = 80
FB = 128


def _pad_rows(N):
    return ((N + NS * FB - 1) // (NS * FB)) * NS * FB


def _sc_pass1(N, E, D):
    per_w = E // (NC * NS)
    nchunk = per_w // CH
    NP = _pad_rows(N)
    rpt = NP // NS
    nflush = rpt // FB

    mesh = plsc.VectorSubcoreMesh(core_axis_name="c", subcore_axis_name="s")

    @functools.partial(
        pl.kernel,
        out_type=(
            jax.ShapeDtypeStruct((NC, NP, D), jnp.float32),
            jax.ShapeDtypeStruct((NC, NP, D), jnp.float32),
            jax.ShapeDtypeStruct((NC, NP, D), jnp.float32),
        ),
        mesh=mesh,
        scratch_types=(
            pltpu.VMEM_SHARED((NP, D), jnp.float32),
            pltpu.VMEM((FB, D), jnp.float32),
            pltpu.VMEM((CH, D), jnp.float32),
            pltpu.VMEM((CH,), jnp.int32),
            pltpu.VMEM((CH,), jnp.int32),
            pltpu.SemaphoreType.DMA,
        ),
    )
    def k(src_ref, dst_ref, x_ref, ea_ref, ax_ref, ae_ref, cnt_ref,
          acc, buf, rows, sidx, didx, sem):
        c = lax.axis_index("c")
        s = lax.axis_index("s")
        ebase = (c * NS + s) * per_w
        rbase = s * rpt
        z16 = jnp.zeros((16,), jnp.float32)

        def zero_buf(i, _):
            for jseg in range(D // 16):
                buf[i, pl.ds(jseg * 16, 16)] = z16
            return 0

        def fill_rows_ones(i, _):
            for jseg in range(D // 16):
                rows[i, pl.ds(jseg * 16, 16)] = z16 + 1.0
            return 0

        def zero_acc():
            lax.fori_loop(0, FB, zero_buf, 0)
            for t in range(nflush):
                pltpu.sync_copy(buf, acc.at[pl.ds(rbase + t * FB, FB)])

        def flush_acc(out_ref):
            for t in range(nflush):
                pltpu.sync_copy(acc.at[pl.ds(rbase + t * FB, FB)], buf)
                pltpu.sync_copy(buf, out_ref.at[c, pl.ds(rbase + t * FB, FB)])

        zero_acc()
        plsc.subcore_barrier()

        def step_x(j, _):
            eb = ebase + j * CH
            pltpu.sync_copy(src_ref.at[pl.ds(eb, CH)], sidx)
            pltpu.sync_copy(dst_ref.at[pl.ds(eb, CH)], didx)
            pltpu.async_copy(x_ref.at[sidx], rows, sem).wait()
            pltpu.sync_copy(rows, acc.at[didx], add=True)
            return 0

        lax.fori_loop(0, nchunk, step_x, 0)
        plsc.subcore_barrier()
        flush_acc(ax_ref)
        zero_acc()
        plsc.subcore_barrier()

        def step_e(j, _):
            eb = ebase + j * CH
            pltpu.sync_copy(dst_ref.at[pl.ds(eb, CH)], didx)
            pltpu.sync_copy(ea_ref.at[pl.ds(eb, CH)], rows)
            pltpu.sync_copy(rows, acc.at[didx], add=True)
            return 0

        lax.fori_loop(0, nchunk, step_e, 0)
        plsc.subcore_barrier()
        flush_acc(ae_ref)
        zero_acc()
        plsc.subcore_barrier()

        lax.fori_loop(0, CH, fill_rows_ones, 0)

        def step_c(j, _):
            eb = ebase + j * CH
            pltpu.sync_copy(dst_ref.at[pl.ds(eb, CH)], didx)
            pltpu.sync_copy(rows, acc.at[didx], add=True)
            return 0

        lax.fori_loop(0, nchunk, step_c, 0)
        plsc.subcore_barrier()
        flush_acc(cnt_ref)

    return k


def _sc_pass2(N, E, D):
    per_w = E // (NC * NS)
    nchunk = per_w // CH
    NP = _pad_rows(N)
    rpt = NP // NS
    nflush = rpt // FB

    mesh = plsc.VectorSubcoreMesh(core_axis_name="c", subcore_axis_name="s")

    @functools.partial(
        pl.kernel,
        out_type=jax.ShapeDtypeStruct((NC, NP, D), jnp.float32),
        mesh=mesh,
        scratch_types=(
            pltpu.VMEM_SHARED((NP, D), jnp.float32),
            pltpu.VMEM((FB, D), jnp.float32),
            pltpu.VMEM((CH, D), jnp.float32),
            pltpu.VMEM((CH,), jnp.int32),
            pltpu.VMEM((CH,), jnp.int32),
            pltpu.SemaphoreType.DMA,
        ),
    )
    def k(src_ref, dst_ref, hm_ref, ah_ref, acc, buf, rows, sidx, didx, sem):
        c = lax.axis_index("c")
        s = lax.axis_index("s")
        ebase = (c * NS + s) * per_w
        rbase = s * rpt
        z16 = jnp.zeros((16,), jnp.float32)

        def zero_buf(i, _):
            for jseg in range(D // 16):
                buf[i, pl.ds(jseg * 16, 16)] = z16
            return 0

        lax.fori_loop(0, FB, zero_buf, 0)
        for t in range(nflush):
            pltpu.sync_copy(buf, acc.at[pl.ds(rbase + t * FB, FB)])
        plsc.subcore_barrier()

        def step(j, _):
            eb = ebase + j * CH
            pltpu.sync_copy(src_ref.at[pl.ds(eb, CH)], sidx)
            pltpu.sync_copy(dst_ref.at[pl.ds(eb, CH)], didx)
            pltpu.async_copy(hm_ref.at[sidx], rows, sem).wait()
            pltpu.sync_copy(rows, acc.at[didx], add=True)
            return 0

        lax.fori_loop(0, nchunk, step, 0)
        plsc.subcore_barrier()

        for t in range(nflush):
            pltpu.sync_copy(acc.at[pl.ds(rbase + t * FB, FB)], buf)
            pltpu.sync_copy(buf, ah_ref.at[c, pl.ds(rbase + t * FB, FB)])

    return k


def _tc1_body(x_ref, ax_ref, ae_ref, cnt_ref, w1m, w1e, w1s, b1m, b1e, b1s,
              w2m, w2e, w2s, b2m, b2e, b2s, hm_ref, p2_ref):
    f32 = jnp.float32
    axs = ax_ref[0] + ax_ref[1]
    aes = ae_ref[0] + ae_ref[1]
    deg = cnt_ref[0, :, 0:1] + cnt_ref[1, :, 0:1]
    pre = (jnp.dot(axs, w1m[...], preferred_element_type=f32)
           + jnp.dot(aes, w1e[...], preferred_element_type=f32)
           + jnp.dot(x_ref[...], w1s[...], preferred_element_type=f32)
           + deg * (b1m[...] + b1e[...]) + b1s[...])
    h = jnp.maximum(pre, 0.0)
    hm_ref[...] = jnp.dot(h, w2m[...], preferred_element_type=f32)
    p2_ref[...] = (jnp.dot(aes, w2e[...], preferred_element_type=f32)
                   + jnp.dot(h, w2s[...], preferred_element_type=f32)
                   + deg * (b2m[...] + b2e[...]) + b2s[...])


def _tc2_body(ah_ref, p2_ref, o_ref):
    o_ref[...] = jnp.maximum(ah_ref[0] + ah_ref[1] + p2_ref[...], 0.0)


def kernel(x, edge_index, edge_attr, W1m, b1m, W1e, b1e, W1s, b1s,
           W2m, b2m, W2e, b2e, W2s, b2s):
    N, D_in = x.shape
    E = edge_index.shape[1]
    D_hid = W1m.shape[1]
    D_out = W2m.shape[1]

    src = edge_index[0]
    dst = edge_index[1]
    ax, ae, cnt = _sc_pass1(N, E, D_in)(src, dst, x, edge_attr)

    B = 1000
    grid = (N // B,)
    full = lambda shape: pl.BlockSpec(shape, lambda i: (0,) * len(shape))
    rowblk = lambda d: pl.BlockSpec((B, d), lambda i: (i, 0))
    partblk = lambda d: pl.BlockSpec((NC, B, d), lambda i: (0, i, 0))

    hm, p2 = pl.pallas_call(
        _tc1_body,
        grid=grid,
        in_specs=[
            rowblk(D_in), partblk(D_in), partblk(D_in), partblk(D_in),
            full((D_in, D_hid)), full((D_in, D_hid)), full((D_in, D_hid)),
            full((1, D_hid)), full((1, D_hid)), full((1, D_hid)),
            full((D_hid, D_out)), full((D_in, D_out)), full((D_hid, D_out)),
            full((1, D_out)), full((1, D_out)), full((1, D_out)),
        ],
        out_specs=[rowblk(D_out), rowblk(D_out)],
        out_shape=[
            jax.ShapeDtypeStruct((N, D_out), jnp.float32),
            jax.ShapeDtypeStruct((N, D_out), jnp.float32),
        ],
    )(x, ax, ae, cnt, W1m, W1e, W1s,
      b1m.reshape(1, -1), b1e.reshape(1, -1), b1s.reshape(1, -1),
      W2m, W2e, W2s,
      b2m.reshape(1, -1), b2e.reshape(1, -1), b2s.reshape(1, -1))

    ah = _sc_pass2(N, E, D_out)(src, dst, hm)

    out = pl.pallas_call(
        _tc2_body,
        grid=grid,
        in_specs=[partblk(D_out), rowblk(D_out)],
        out_specs=rowblk(D_out),
        out_shape=jax.ShapeDtypeStruct((N, D_out), jnp.float32),
    )(ah, p2)
    return out
